# Initial kernel scaffold; baseline (speedup 1.0000x reference)
import jax
import jax.numpy as jnp
from jax.experimental import pallas as pl


def kernel(x, edge_index, W1, b1, W2, b2, W3, b3):
    raise NotImplementedError("write your pallas kernel here")


# speedup vs baseline: 192.1345x; 192.1345x over previous
import functools

import jax
import jax.numpy as jnp
from jax import lax
from jax.experimental import pallas as pl
from jax.experimental.pallas import tpu as pltpu
from jax.experimental.pallas import tpu_sc as plsc

f32 = jnp.float32
i32 = jnp.int32

NN = 100000
EE = 6400000
NC = 2
NS = 16
NW = NC * NS
NP = 102400
NSL = NP // NS
EW = EE // NW
CE = 10000
E2 = EE // 2
PW = E2 // NW
CP = 4000
V16 = 16


def _mesh():
    return plsc.VectorSubcoreMesh(core_axis_name="c", subcore_axis_name="s")


def _wid():
    return lax.axis_index("c") * NS + lax.axis_index("s")


def _fill(ref, n, val):
    def body(i, _):
        ref[pl.ds(i * V16, V16)] = jnp.full((V16,), val, f32)
        return 0
    lax.fori_loop(0, n // V16, body, 0)


def _rsqrt16(d):
    ii = plsc.bitcast(d, i32)
    seed = jnp.full((V16,), 0x5F3759DF, i32) - lax.shift_right_arithmetic(
        ii, jnp.full((V16,), 1, i32))
    y = plsc.bitcast(seed, f32)
    for _ in range(3):
        y = y * (1.5 - 0.5 * d * y * y)
    return y


def _deg_body(dst_hbm, degp_hbm, acc_sh, idxb, oneb, zb):
    c = lax.axis_index("c")
    s = lax.axis_index("s")
    sl = pl.ds(s * NSL, NSL)
    _fill(zb, NSL, 0.0)
    pltpu.sync_copy(zb, acc_sh.at[sl])
    _fill(oneb, CE, 1.0)
    plsc.subcore_barrier()
    base = _wid() * EW

    def chunk(i, _):
        pltpu.sync_copy(dst_hbm.at[pl.ds(base + i * CE, CE)], idxb)
        pltpu.sync_copy(oneb, acc_sh.at[idxb], add=True)
        return 0
    lax.fori_loop(0, EW // CE, chunk, 0)
    plsc.subcore_barrier()
    pltpu.sync_copy(acc_sh.at[sl], degp_hbm.at[c, sl])


_deg = pl.kernel(
    _deg_body,
    out_type=jax.ShapeDtypeStruct((NC, NP), f32),
    mesh=_mesh(),
    scratch_types=[
        pltpu.VMEM_SHARED((NP,), f32),
        pltpu.VMEM((CE,), i32),
        pltpu.VMEM((CE,), f32),
        pltpu.VMEM((NSL,), f32),
    ],
)


def _p1_body(src_hbm, dst_hbm, xt_hbm, degp_hbm, par_hbm,
             dinv_hbm, g1_hbm, s1p_hbm,
             tab_sh, acc_sh, pb, b0, b1, q0, q1, q2, q3, db, gb, ib, jb, vb):
    c = lax.axis_index("c")
    s = lax.axis_index("s")
    sl = pl.ds(s * NSL, NSL)
    pltpu.sync_copy(par_hbm, pb)
    pltpu.sync_copy(degp_hbm.at[0, sl], b0)
    pltpu.sync_copy(degp_hbm.at[1, sl], b1)
    pltpu.sync_copy(xt_hbm.at[0, sl], q0)
    pltpu.sync_copy(xt_hbm.at[1, sl], q1)
    pltpu.sync_copy(xt_hbm.at[2, sl], q2)
    pltpu.sync_copy(xt_hbm.at[3, sl], q3)
    pv = pb[pl.ds(0, 16)]
    w0 = pv[0]
    w1 = pv[1]
    w2 = pv[2]
    w3 = pv[3]

    def nodes(i, _):
        dd = pl.ds(i * V16, V16)
        deg = b0[dd] + b1[dd] + 1.0
        y = _rsqrt16(deg)
        t0 = q0[dd] * w0 + q1[dd] * w1 + q2[dd] * w2 + q3[dd] * w3
        db[dd] = y
        gb[dd] = y * t0
        return 0
    lax.fori_loop(0, NSL // V16, nodes, 0)
    pltpu.sync_copy(db, dinv_hbm.at[sl])
    pltpu.sync_copy(gb, g1_hbm.at[sl])
    pltpu.sync_copy(gb, tab_sh.at[sl])
    _fill(b0, NSL, 0.0)
    pltpu.sync_copy(b0, acc_sh.at[sl])
    plsc.subcore_barrier()
    base = _wid() * EW

    def chunk(i, _):
        off = base + i * CE
        pltpu.sync_copy(src_hbm.at[pl.ds(off, CE)], ib)
        pltpu.sync_copy(dst_hbm.at[pl.ds(off, CE)], jb)
        pltpu.sync_copy(tab_sh.at[ib], vb)
        pltpu.sync_copy(vb, acc_sh.at[jb], add=True)
        return 0
    lax.fori_loop(0, EW // CE, chunk, 0)
    plsc.subcore_barrier()
    pltpu.sync_copy(acc_sh.at[sl], s1p_hbm.at[c, sl])


_k1 = pl.kernel(
    _p1_body,
    out_type=(jax.ShapeDtypeStruct((NP,), f32),
              jax.ShapeDtypeStruct((NP,), f32),
              jax.ShapeDtypeStruct((NC, NP), f32)),
    mesh=_mesh(),
    compiler_params=pltpu.CompilerParams(needs_layout_passes=False),
    scratch_types=[
        pltpu.VMEM_SHARED((NP,), f32),
        pltpu.VMEM_SHARED((NP,), f32),
        pltpu.VMEM((16,), f32),
        pltpu.VMEM((NSL,), f32),
        pltpu.VMEM((NSL,), f32),
        pltpu.VMEM((NSL,), f32),
        pltpu.VMEM((NSL,), f32),
        pltpu.VMEM((NSL,), f32),
        pltpu.VMEM((NSL,), f32),
        pltpu.VMEM((NSL,), f32),
        pltpu.VMEM((NSL,), f32),
        pltpu.VMEM((CE,), i32),
        pltpu.VMEM((CE,), i32),
        pltpu.VMEM((CE,), f32),
    ],
)


def _p2_body(src_hbm, dst_hbm, dinv_hbm, g1_hbm, s1p_hbm, par_hbm,
             g2_hbm, s2p_hbm,
             tab_sh, acc_sh, pb, b0, b1, dq, gq, gb, ib, jb, vb):
    c = lax.axis_index("c")
    s = lax.axis_index("s")
    sl = pl.ds(s * NSL, NSL)
    pltpu.sync_copy(par_hbm, pb)
    pltpu.sync_copy(s1p_hbm.at[0, sl], b0)
    pltpu.sync_copy(s1p_hbm.at[1, sl], b1)
    pltpu.sync_copy(dinv_hbm.at[sl], dq)
    pltpu.sync_copy(g1_hbm.at[sl], gq)
    pv = pb[pl.ds(0, 16)]
    bias1 = pv[4]

    def nodes(i, _):
        dd = pl.ds(i * V16, V16)
        d = dq[dd]
        h1 = jnp.maximum(d * (b0[dd] + b1[dd] + gq[dd]) + bias1, 0.0)
        gb[dd] = d * h1
        return 0
    lax.fori_loop(0, NSL // V16, nodes, 0)
    pltpu.sync_copy(gb, g2_hbm.at[sl])
    pltpu.sync_copy(gb, tab_sh.at[sl])
    _fill(b0, NSL, 0.0)
    pltpu.sync_copy(b0, acc_sh.at[sl])
    plsc.subcore_barrier()
    base = _wid() * EW

    def chunk(i, _):
        off = base + i * CE
        pltpu.sync_copy(src_hbm.at[pl.ds(off, CE)], ib)
        pltpu.sync_copy(dst_hbm.at[pl.ds(off, CE)], jb)
        pltpu.sync_copy(tab_sh.at[ib], vb)
        pltpu.sync_copy(vb, acc_sh.at[jb], add=True)
        return 0
    lax.fori_loop(0, EW // CE, chunk, 0)
    plsc.subcore_barrier()
    pltpu.sync_copy(acc_sh.at[sl], s2p_hbm.at[c, sl])


_k2 = pl.kernel(
    _p2_body,
    out_type=(jax.ShapeDtypeStruct((NP,), f32),
              jax.ShapeDtypeStruct((NC, NP), f32)),
    mesh=_mesh(),
    scratch_types=[
        pltpu.VMEM_SHARED((NP,), f32),
        pltpu.VMEM_SHARED((NP,), f32),
        pltpu.VMEM((16,), f32),
        pltpu.VMEM((NSL,), f32),
        pltpu.VMEM((NSL,), f32),
        pltpu.VMEM((NSL,), f32),
        pltpu.VMEM((NSL,), f32),
        pltpu.VMEM((NSL,), f32),
        pltpu.VMEM((CE,), i32),
        pltpu.VMEM((CE,), i32),
        pltpu.VMEM((CE,), f32),
    ],
)


def _p3_body(src_hbm, dst_hbm, dinv_hbm, g2_hbm, s2p_hbm, par_hbm,
             g3a_hbm, g3b_hbm, s3a_hbm, s3b_hbm,
             taba_sh, tabb_sh, acca_sh, accb_sh,
             pb, b0, b1, dq, gq, ga, gb2, ib, jb, va, vb):
    c = lax.axis_index("c")
    s = lax.axis_index("s")
    sl = pl.ds(s * NSL, NSL)
    pltpu.sync_copy(par_hbm, pb)
    pltpu.sync_copy(s2p_hbm.at[0, sl], b0)
    pltpu.sync_copy(s2p_hbm.at[1, sl], b1)
    pltpu.sync_copy(dinv_hbm.at[sl], dq)
    pltpu.sync_copy(g2_hbm.at[sl], gq)
    pv = pb[pl.ds(0, 16)]
    w2a = pv[5]
    w2b = pv[6]
    b2a = pv[7]
    b2b = pv[8]

    def nodes(i, _):
        dd = pl.ds(i * V16, V16)
        d = dq[dd]
        u = d * (b0[dd] + b1[dd] + gq[dd])
        h2a = jnp.maximum(u * w2a + b2a, 0.0)
        h2b = jnp.maximum(u * w2b + b2b, 0.0)
        ga[dd] = d * h2a
        gb2[dd] = d * h2b
        return 0
    lax.fori_loop(0, NSL // V16, nodes, 0)
    pltpu.sync_copy(ga, g3a_hbm.at[sl])
    pltpu.sync_copy(gb2, g3b_hbm.at[sl])
    pltpu.sync_copy(ga, taba_sh.at[sl])
    pltpu.sync_copy(gb2, tabb_sh.at[sl])
    _fill(b0, NSL, 0.0)
    pltpu.sync_copy(b0, acca_sh.at[sl])
    pltpu.sync_copy(b0, accb_sh.at[sl])
    plsc.subcore_barrier()
    base = _wid() * EW

    def chunk(i, _):
        off = base + i * CE
        pltpu.sync_copy(src_hbm.at[pl.ds(off, CE)], ib)
        pltpu.sync_copy(dst_hbm.at[pl.ds(off, CE)], jb)
        pltpu.sync_copy(taba_sh.at[ib], va)
        pltpu.sync_copy(tabb_sh.at[ib], vb)
        pltpu.sync_copy(va, acca_sh.at[jb], add=True)
        pltpu.sync_copy(vb, accb_sh.at[jb], add=True)
        return 0
    lax.fori_loop(0, EW // CE, chunk, 0)
    plsc.subcore_barrier()
    pltpu.sync_copy(acca_sh.at[sl], s3a_hbm.at[c, sl])
    pltpu.sync_copy(accb_sh.at[sl], s3b_hbm.at[c, sl])


_k3 = pl.kernel(
    _p3_body,
    out_type=(jax.ShapeDtypeStruct((NP,), f32),
              jax.ShapeDtypeStruct((NP,), f32),
              jax.ShapeDtypeStruct((NC, NP), f32),
              jax.ShapeDtypeStruct((NC, NP), f32)),
    mesh=_mesh(),
    scratch_types=[
        pltpu.VMEM_SHARED((NP,), f32),
        pltpu.VMEM_SHARED((NP,), f32),
        pltpu.VMEM_SHARED((NP,), f32),
        pltpu.VMEM_SHARED((NP,), f32),
        pltpu.VMEM((16,), f32),
        pltpu.VMEM((NSL,), f32),
        pltpu.VMEM((NSL,), f32),
        pltpu.VMEM((NSL,), f32),
        pltpu.VMEM((NSL,), f32),
        pltpu.VMEM((NSL,), f32),
        pltpu.VMEM((NSL,), f32),
        pltpu.VMEM((CE,), i32),
        pltpu.VMEM((CE,), i32),
        pltpu.VMEM((CE,), f32),
        pltpu.VMEM((CE,), f32),
    ],
)


def _ro_body(src_hbm, dst_hbm, dinv_hbm, g3a_hbm, g3b_hbm,
             s3a_hbm, s3b_hbm, par_hbm, o_hbm,
             tva_sh, tvb_sh, tr0_sh, tr1_sh, tt_sh,
             pb, dq, a0, a1, e0, e1, gqa, gqb,
             ib, jb, fva, fvb, fr0, fr1, ft, sb):
    s = lax.axis_index("s")
    sl = pl.ds(s * NSL, NSL)
    pltpu.sync_copy(par_hbm, pb)
    pltpu.sync_copy(s3a_hbm.at[0, sl], a0)
    pltpu.sync_copy(s3a_hbm.at[1, sl], a1)
    pltpu.sync_copy(s3b_hbm.at[0, sl], e0)
    pltpu.sync_copy(s3b_hbm.at[1, sl], e1)
    pltpu.sync_copy(dinv_hbm.at[sl], dq)
    pltpu.sync_copy(g3a_hbm.at[sl], gqa)
    pltpu.sync_copy(g3b_hbm.at[sl], gqb)
    pv = pb[pl.ds(0, 16)]
    g00 = pv[9]
    g01 = pv[10]
    g10 = pv[11]
    g11 = pv[12]
    c0 = pv[13]
    c1 = pv[14]
    k0 = pv[15]

    def nodes(i, _):
        dd = pl.ds(i * V16, V16)
        d = dq[dd]
        va = d * (a0[dd] + a1[dd] + gqa[dd])
        vb = d * (e0[dd] + e1[dd] + gqb[dd])
        a0[dd] = va
        e0[dd] = vb
        a1[dd] = g00 * va + g01 * vb + c0
        e1[dd] = g10 * va + g11 * vb + c1
        gqa[dd] = c0 * va + c1 * vb + k0
        return 0
    lax.fori_loop(0, NSL // V16, nodes, 0)
    pltpu.sync_copy(a0, tva_sh.at[sl])
    pltpu.sync_copy(e0, tvb_sh.at[sl])
    pltpu.sync_copy(a1, tr0_sh.at[sl])
    pltpu.sync_copy(e1, tr1_sh.at[sl])
    pltpu.sync_copy(gqa, tt_sh.at[sl])
    plsc.subcore_barrier()
    base = _wid() * PW

    def half(off):
        pltpu.sync_copy(src_hbm.at[pl.ds(off, CP)], ib)
        pltpu.sync_copy(dst_hbm.at[pl.ds(off, CP)], jb)
        pltpu.sync_copy(tva_sh.at[ib], fva)
        pltpu.sync_copy(tvb_sh.at[ib], fvb)
        pltpu.sync_copy(tr0_sh.at[jb], fr0)
        pltpu.sync_copy(tr1_sh.at[jb], fr1)
        pltpu.sync_copy(tt_sh.at[jb], ft)

    def chunk(i, _):
        off = base + i * CP
        half(off)

        def dot1(j, _):
            dd = pl.ds(j * V16, V16)
            sb[dd] = fva[dd] * fr0[dd] + fvb[dd] * fr1[dd] + ft[dd]
            return 0
        lax.fori_loop(0, CP // V16, dot1, 0)
        half(off + E2)

        def dot2(j, _):
            dd = pl.ds(j * V16, V16)
            sv = 0.5 * (sb[dd] + fva[dd] * fr0[dd] + fvb[dd] * fr1[dd] + ft[dd])
            sb[dd] = 1.0 / (1.0 + jnp.exp(-sv))
            return 0
        lax.fori_loop(0, CP // V16, dot2, 0)
        pltpu.sync_copy(sb, o_hbm.at[pl.ds(off, CP)])
        return 0
    lax.fori_loop(0, PW // CP, chunk, 0)


_k4 = pl.kernel(
    _ro_body,
    out_type=jax.ShapeDtypeStruct((E2,), f32),
    mesh=_mesh(),
    scratch_types=[
        pltpu.VMEM_SHARED((NP,), f32),
        pltpu.VMEM_SHARED((NP,), f32),
        pltpu.VMEM_SHARED((NP,), f32),
        pltpu.VMEM_SHARED((NP,), f32),
        pltpu.VMEM_SHARED((NP,), f32),
        pltpu.VMEM((16,), f32),
        pltpu.VMEM((NSL,), f32),
        pltpu.VMEM((NSL,), f32),
        pltpu.VMEM((NSL,), f32),
        pltpu.VMEM((NSL,), f32),
        pltpu.VMEM((NSL,), f32),
        pltpu.VMEM((NSL,), f32),
        pltpu.VMEM((NSL,), f32),
        pltpu.VMEM((CP,), i32),
        pltpu.VMEM((CP,), i32),
        pltpu.VMEM((CP,), f32),
        pltpu.VMEM((CP,), f32),
        pltpu.VMEM((CP,), f32),
        pltpu.VMEM((CP,), f32),
        pltpu.VMEM((CP,), f32),
        pltpu.VMEM((CP,), f32),
    ],
)


def kernel(x, edge_index, W1, b1, W2, b2, W3, b3):
    src = edge_index[0]
    dst = edge_index[1]
    xt = jnp.zeros((4, NP), f32).at[:, :NN].set(x.T)
    G = W3 @ W3.T
    cvec = W3 @ b3
    k0 = jnp.dot(b3, b3)
    params = jnp.concatenate(
        [W1[:, 0], b1, W2[0], b2, G.ravel(), cvec, k0[None]]).astype(f32)
    degp = _deg(dst)
    dinv, g1, s1p = _k1(src, dst, xt, degp, params)
    g2, s2p = _k2(src, dst, dinv, g1, s1p, params)
    g3a, g3b, s3a, s3b = _k3(src, dst, dinv, g2, s2p, params)
    o = _k4(src, dst, dinv, g3a, g3b, s3a, s3b, params)
    return o[:, None]


# speedup vs baseline: 213.6582x; 1.1120x over previous
import functools

import jax
import jax.numpy as jnp
from jax import lax
from jax.experimental import pallas as pl
from jax.experimental.pallas import tpu as pltpu
from jax.experimental.pallas import tpu_sc as plsc

f32 = jnp.float32
i32 = jnp.int32

NN = 100000
EE = 6400000
NC = 2
NS = 16
NW = NC * NS
NP = 102400
NSL = NP // NS
EW = EE // NW
CE = 10000
E2 = EE // 2
PW = E2 // NW
CP = 4000
V16 = 16


def _mesh():
    return plsc.VectorSubcoreMesh(core_axis_name="c", subcore_axis_name="s")


def _wid():
    return lax.axis_index("c") * NS + lax.axis_index("s")


def _fill(ref, n, val):
    def body(i, _):
        ref[pl.ds(i * V16, V16)] = jnp.full((V16,), val, f32)
        return 0
    lax.fori_loop(0, n // V16, body, 0)


def _rsqrt16(d):
    ii = plsc.bitcast(d, i32)
    seed = jnp.full((V16,), 0x5F3759DF, i32) - lax.shift_right_arithmetic(
        ii, jnp.full((V16,), 1, i32))
    y = plsc.bitcast(seed, f32)
    for _ in range(3):
        y = y * (1.5 - 0.5 * d * y * y)
    return y


def _deg_body(dst_hbm, degp_hbm, acc_sh, idxb, oneb, zb):
    c = lax.axis_index("c")
    s = lax.axis_index("s")
    sl = pl.ds(s * NSL, NSL)
    _fill(zb, NSL, 0.0)
    pltpu.sync_copy(zb, acc_sh.at[sl])
    _fill(oneb, CE, 1.0)
    plsc.subcore_barrier()
    base = _wid() * EW

    def chunk(i, _):
        pltpu.sync_copy(dst_hbm.at[pl.ds(base + i * CE, CE)], idxb)
        pltpu.sync_copy(oneb, acc_sh.at[idxb], add=True)
        return 0
    lax.fori_loop(0, EW // CE, chunk, 0)
    plsc.subcore_barrier()
    pltpu.sync_copy(acc_sh.at[sl], degp_hbm.at[c, sl])


_deg = pl.kernel(
    _deg_body,
    out_type=jax.ShapeDtypeStruct((NC, NP), f32),
    mesh=_mesh(),
    compiler_params=pltpu.CompilerParams(needs_layout_passes=False),
    scratch_types=[
        pltpu.VMEM_SHARED((NP,), f32),
        pltpu.VMEM((CE,), i32),
        pltpu.VMEM((CE,), f32),
        pltpu.VMEM((NSL,), f32),
    ],
)


def _p1_body(src_hbm, dst_hbm, xt_hbm, degp_hbm, par_hbm,
             dinv_hbm, g1_hbm, s1p_hbm,
             tab_sh, acc_sh, pb, b0, b1, q0, q1, q2, q3, db, gb, ib, jb, vb):
    c = lax.axis_index("c")
    s = lax.axis_index("s")
    sl = pl.ds(s * NSL, NSL)
    pltpu.sync_copy(par_hbm, pb)
    pltpu.sync_copy(degp_hbm.at[0, sl], b0)
    pltpu.sync_copy(degp_hbm.at[1, sl], b1)
    pltpu.sync_copy(xt_hbm.at[0, sl], q0)
    pltpu.sync_copy(xt_hbm.at[1, sl], q1)
    pltpu.sync_copy(xt_hbm.at[2, sl], q2)
    pltpu.sync_copy(xt_hbm.at[3, sl], q3)
    pv = pb[pl.ds(0, 16)]
    w0 = pv[0]
    w1 = pv[1]
    w2 = pv[2]
    w3 = pv[3]

    def nodes(i, _):
        dd = pl.ds(i * V16, V16)
        deg = b0[dd] + b1[dd] + 1.0
        y = _rsqrt16(deg)
        t0 = q0[dd] * w0 + q1[dd] * w1 + q2[dd] * w2 + q3[dd] * w3
        db[dd] = y
        gb[dd] = y * t0
        return 0
    lax.fori_loop(0, NSL // V16, nodes, 0)
    pltpu.sync_copy(db, dinv_hbm.at[sl])
    pltpu.sync_copy(gb, g1_hbm.at[sl])
    pltpu.sync_copy(gb, tab_sh.at[sl])
    _fill(b0, NSL, 0.0)
    pltpu.sync_copy(b0, acc_sh.at[sl])
    plsc.subcore_barrier()
    base = _wid() * EW

    def chunk(i, _):
        off = base + i * CE
        pltpu.sync_copy(src_hbm.at[pl.ds(off, CE)], ib)
        pltpu.sync_copy(dst_hbm.at[pl.ds(off, CE)], jb)
        pltpu.sync_copy(tab_sh.at[ib], vb)
        pltpu.sync_copy(vb, acc_sh.at[jb], add=True)
        return 0
    lax.fori_loop(0, EW // CE, chunk, 0)
    plsc.subcore_barrier()
    pltpu.sync_copy(acc_sh.at[sl], s1p_hbm.at[c, sl])


_k1 = pl.kernel(
    _p1_body,
    out_type=(jax.ShapeDtypeStruct((NP,), f32),
              jax.ShapeDtypeStruct((NP,), f32),
              jax.ShapeDtypeStruct((NC, NP), f32)),
    mesh=_mesh(),
    compiler_params=pltpu.CompilerParams(needs_layout_passes=False),
    scratch_types=[
        pltpu.VMEM_SHARED((NP,), f32),
        pltpu.VMEM_SHARED((NP,), f32),
        pltpu.VMEM((16,), f32),
        pltpu.VMEM((NSL,), f32),
        pltpu.VMEM((NSL,), f32),
        pltpu.VMEM((NSL,), f32),
        pltpu.VMEM((NSL,), f32),
        pltpu.VMEM((NSL,), f32),
        pltpu.VMEM((NSL,), f32),
        pltpu.VMEM((NSL,), f32),
        pltpu.VMEM((NSL,), f32),
        pltpu.VMEM((CE,), i32),
        pltpu.VMEM((CE,), i32),
        pltpu.VMEM((CE,), f32),
    ],
)


def _p2_body(src_hbm, dst_hbm, dinv_hbm, g1_hbm, s1p_hbm, par_hbm,
             g2_hbm, s2p_hbm,
             tab_sh, acc_sh, pb, b0, b1, dq, gq, gb, ib, jb, vb):
    c = lax.axis_index("c")
    s = lax.axis_index("s")
    sl = pl.ds(s * NSL, NSL)
    pltpu.sync_copy(par_hbm, pb)
    pltpu.sync_copy(s1p_hbm.at[0, sl], b0)
    pltpu.sync_copy(s1p_hbm.at[1, sl], b1)
    pltpu.sync_copy(dinv_hbm.at[sl], dq)
    pltpu.sync_copy(g1_hbm.at[sl], gq)
    pv = pb[pl.ds(0, 16)]
    bias1 = pv[4]

    def nodes(i, _):
        dd = pl.ds(i * V16, V16)
        d = dq[dd]
        h1 = jnp.maximum(d * (b0[dd] + b1[dd] + gq[dd]) + bias1, 0.0)
        gb[dd] = d * h1
        return 0
    lax.fori_loop(0, NSL // V16, nodes, 0)
    pltpu.sync_copy(gb, g2_hbm.at[sl])
    pltpu.sync_copy(gb, tab_sh.at[sl])
    _fill(b0, NSL, 0.0)
    pltpu.sync_copy(b0, acc_sh.at[sl])
    plsc.subcore_barrier()
    base = _wid() * EW

    def chunk(i, _):
        off = base + i * CE
        pltpu.sync_copy(src_hbm.at[pl.ds(off, CE)], ib)
        pltpu.sync_copy(dst_hbm.at[pl.ds(off, CE)], jb)
        pltpu.sync_copy(tab_sh.at[ib], vb)
        pltpu.sync_copy(vb, acc_sh.at[jb], add=True)
        return 0
    lax.fori_loop(0, EW // CE, chunk, 0)
    plsc.subcore_barrier()
    pltpu.sync_copy(acc_sh.at[sl], s2p_hbm.at[c, sl])


_k2 = pl.kernel(
    _p2_body,
    out_type=(jax.ShapeDtypeStruct((NP,), f32),
              jax.ShapeDtypeStruct((NC, NP), f32)),
    mesh=_mesh(),
    compiler_params=pltpu.CompilerParams(needs_layout_passes=False),
    scratch_types=[
        pltpu.VMEM_SHARED((NP,), f32),
        pltpu.VMEM_SHARED((NP,), f32),
        pltpu.VMEM((16,), f32),
        pltpu.VMEM((NSL,), f32),
        pltpu.VMEM((NSL,), f32),
        pltpu.VMEM((NSL,), f32),
        pltpu.VMEM((NSL,), f32),
        pltpu.VMEM((NSL,), f32),
        pltpu.VMEM((CE,), i32),
        pltpu.VMEM((CE,), i32),
        pltpu.VMEM((CE,), f32),
    ],
)


def _p3_body(src_hbm, dst_hbm, dinv_hbm, g2_hbm, s2p_hbm, par_hbm,
             g3a_hbm, g3b_hbm, s3a_hbm, s3b_hbm,
             tpk_sh, acca_sh, accb_sh,
             pb, b0, b1, dq, gq, ga, gb2, pkb, ib, jb, wb, va, vb):
    c = lax.axis_index("c")
    s = lax.axis_index("s")
    sl = pl.ds(s * NSL, NSL)
    pltpu.sync_copy(par_hbm, pb)
    pltpu.sync_copy(s2p_hbm.at[0, sl], b0)
    pltpu.sync_copy(s2p_hbm.at[1, sl], b1)
    pltpu.sync_copy(dinv_hbm.at[sl], dq)
    pltpu.sync_copy(g2_hbm.at[sl], gq)
    pv = pb[pl.ds(0, 16)]
    w2a = pv[5]
    w2b = pv[6]
    b2a = pv[7]
    b2b = pv[8]

    def nodes(i, _):
        dd = pl.ds(i * V16, V16)
        d = dq[dd]
        u = d * (b0[dd] + b1[dd] + gq[dd])
        xa = d * jnp.maximum(u * w2a + b2a, 0.0)
        xb = d * jnp.maximum(u * w2b + b2b, 0.0)
        ga[dd] = xa
        gb2[dd] = xb
        pkb[dd] = plsc.bitcast(
            plsc.pack(xa, xb, format=plsc.PackFormat.INTERLEAVED), i32)
        return 0
    lax.fori_loop(0, NSL // V16, nodes, 0)
    pltpu.sync_copy(ga, g3a_hbm.at[sl])
    pltpu.sync_copy(gb2, g3b_hbm.at[sl])
    pltpu.sync_copy(pkb, tpk_sh.at[sl])
    _fill(b0, NSL, 0.0)
    pltpu.sync_copy(b0, acca_sh.at[sl])
    pltpu.sync_copy(b0, accb_sh.at[sl])
    plsc.subcore_barrier()
    base = _wid() * EW

    def chunk(i, _):
        off = base + i * CE
        pltpu.sync_copy(src_hbm.at[pl.ds(off, CE)], ib)
        pltpu.sync_copy(dst_hbm.at[pl.ds(off, CE)], jb)
        pltpu.sync_copy(tpk_sh.at[ib], wb)

        def unpk(j, _):
            dd = pl.ds(j * V16, V16)
            xa, xb = plsc.unpack(plsc.bitcast(wb[dd], jnp.bfloat16),
                                 format=plsc.PackFormat.INTERLEAVED)
            va[dd] = xa
            vb[dd] = xb
            return 0
        lax.fori_loop(0, CE // V16, unpk, 0)
        pltpu.sync_copy(va, acca_sh.at[jb], add=True)
        pltpu.sync_copy(vb, accb_sh.at[jb], add=True)
        return 0
    lax.fori_loop(0, EW // CE, chunk, 0)
    plsc.subcore_barrier()
    pltpu.sync_copy(acca_sh.at[sl], s3a_hbm.at[c, sl])
    pltpu.sync_copy(accb_sh.at[sl], s3b_hbm.at[c, sl])


_k3 = pl.kernel(
    _p3_body,
    out_type=(jax.ShapeDtypeStruct((NP,), f32),
              jax.ShapeDtypeStruct((NP,), f32),
              jax.ShapeDtypeStruct((NC, NP), f32),
              jax.ShapeDtypeStruct((NC, NP), f32)),
    mesh=_mesh(),
    compiler_params=pltpu.CompilerParams(needs_layout_passes=False),
    scratch_types=[
        pltpu.VMEM_SHARED((NP,), i32),
        pltpu.VMEM_SHARED((NP,), f32),
        pltpu.VMEM_SHARED((NP,), f32),
        pltpu.VMEM((16,), f32),
        pltpu.VMEM((NSL,), f32),
        pltpu.VMEM((NSL,), f32),
        pltpu.VMEM((NSL,), f32),
        pltpu.VMEM((NSL,), f32),
        pltpu.VMEM((NSL,), f32),
        pltpu.VMEM((NSL,), f32),
        pltpu.VMEM((NSL,), i32),
        pltpu.VMEM((CE,), i32),
        pltpu.VMEM((CE,), i32),
        pltpu.VMEM((CE,), i32),
        pltpu.VMEM((CE,), f32),
        pltpu.VMEM((CE,), f32),
    ],
)


def _ro_body(src_hbm, dst_hbm, dinv_hbm, g3a_hbm, g3b_hbm,
             s3a_hbm, s3b_hbm, par_hbm, o_hbm,
             tza_sh, tzr_sh, tt_sh,
             pb, dq, a0, a1, e0, e1, gqa, gqb, zab, zrb,
             ib, jb, wa, wr, ft, sb):
    s = lax.axis_index("s")
    sl = pl.ds(s * NSL, NSL)
    pltpu.sync_copy(par_hbm, pb)
    pltpu.sync_copy(s3a_hbm.at[0, sl], a0)
    pltpu.sync_copy(s3a_hbm.at[1, sl], a1)
    pltpu.sync_copy(s3b_hbm.at[0, sl], e0)
    pltpu.sync_copy(s3b_hbm.at[1, sl], e1)
    pltpu.sync_copy(dinv_hbm.at[sl], dq)
    pltpu.sync_copy(g3a_hbm.at[sl], gqa)
    pltpu.sync_copy(g3b_hbm.at[sl], gqb)
    pv = pb[pl.ds(0, 16)]
    g00 = pv[9]
    g01 = pv[10]
    g10 = pv[11]
    g11 = pv[12]
    c0 = pv[13]
    c1 = pv[14]
    k0 = pv[15]

    def nodes(i, _):
        dd = pl.ds(i * V16, V16)
        d = dq[dd]
        va = d * (a0[dd] + a1[dd] + gqa[dd])
        vb = d * (e0[dd] + e1[dd] + gqb[dd])
        zab[dd] = plsc.bitcast(
            plsc.pack(va, vb, format=plsc.PackFormat.INTERLEAVED), i32)
        zrb[dd] = plsc.bitcast(
            plsc.pack(g00 * va + g01 * vb + c0, g10 * va + g11 * vb + c1,
                      format=plsc.PackFormat.INTERLEAVED), i32)
        gqa[dd] = c0 * va + c1 * vb + k0
        return 0
    lax.fori_loop(0, NSL // V16, nodes, 0)
    pltpu.sync_copy(zab, tza_sh.at[sl])
    pltpu.sync_copy(zrb, tzr_sh.at[sl])
    pltpu.sync_copy(gqa, tt_sh.at[sl])
    plsc.subcore_barrier()
    base = _wid() * PW

    def half(off):
        pltpu.sync_copy(src_hbm.at[pl.ds(off, CP)], ib)
        pltpu.sync_copy(dst_hbm.at[pl.ds(off, CP)], jb)
        pltpu.sync_copy(tza_sh.at[ib], wa)
        pltpu.sync_copy(tzr_sh.at[jb], wr)
        pltpu.sync_copy(tt_sh.at[jb], ft)

    def chunk(i, _):
        off = base + i * CP
        half(off)

        def dot1(j, _):
            dd = pl.ds(j * V16, V16)
            va, vb = plsc.unpack(plsc.bitcast(wa[dd], jnp.bfloat16),
                                 format=plsc.PackFormat.INTERLEAVED)
            r0, r1 = plsc.unpack(plsc.bitcast(wr[dd], jnp.bfloat16),
                                 format=plsc.PackFormat.INTERLEAVED)
            sb[dd] = va * r0 + vb * r1 + ft[dd]
            return 0
        lax.fori_loop(0, CP // V16, dot1, 0)
        half(off + E2)

        def dot2(j, _):
            dd = pl.ds(j * V16, V16)
            va, vb = plsc.unpack(plsc.bitcast(wa[dd], jnp.bfloat16),
                                 format=plsc.PackFormat.INTERLEAVED)
            r0, r1 = plsc.unpack(plsc.bitcast(wr[dd], jnp.bfloat16),
                                 format=plsc.PackFormat.INTERLEAVED)
            sv = 0.5 * (sb[dd] + va * r0 + vb * r1 + ft[dd])
            sb[dd] = 1.0 / (1.0 + jnp.exp(-sv))
            return 0
        lax.fori_loop(0, CP // V16, dot2, 0)
        pltpu.sync_copy(sb, o_hbm.at[pl.ds(off, CP)])
        return 0
    lax.fori_loop(0, PW // CP, chunk, 0)


_k4 = pl.kernel(
    _ro_body,
    out_type=jax.ShapeDtypeStruct((E2,), f32),
    mesh=_mesh(),
    compiler_params=pltpu.CompilerParams(needs_layout_passes=False),
    scratch_types=[
        pltpu.VMEM_SHARED((NP,), i32),
        pltpu.VMEM_SHARED((NP,), i32),
        pltpu.VMEM_SHARED((NP,), f32),
        pltpu.VMEM((16,), f32),
        pltpu.VMEM((NSL,), f32),
        pltpu.VMEM((NSL,), f32),
        pltpu.VMEM((NSL,), f32),
        pltpu.VMEM((NSL,), f32),
        pltpu.VMEM((NSL,), f32),
        pltpu.VMEM((NSL,), f32),
        pltpu.VMEM((NSL,), f32),
        pltpu.VMEM((NSL,), i32),
        pltpu.VMEM((NSL,), i32),
        pltpu.VMEM((CP,), i32),
        pltpu.VMEM((CP,), i32),
        pltpu.VMEM((CP,), i32),
        pltpu.VMEM((CP,), i32),
        pltpu.VMEM((CP,), f32),
        pltpu.VMEM((CP,), f32),
    ],
)


def kernel(x, edge_index, W1, b1, W2, b2, W3, b3):
    src = edge_index[0]
    dst = edge_index[1]
    xt = jnp.zeros((4, NP), f32).at[:, :NN].set(x.T)
    G = W3 @ W3.T
    cvec = W3 @ b3
    k0 = jnp.dot(b3, b3)
    params = jnp.concatenate(
        [W1[:, 0], b1, W2[0], b2, G.ravel(), cvec, k0[None]]).astype(f32)
    degp = _deg(dst)
    dinv, g1, s1p = _k1(src, dst, xt, degp, params)
    g2, s2p = _k2(src, dst, dinv, g1, s1p, params)
    g3a, g3b, s3a, s3b = _k3(src, dst, dinv, g2, s2p, params)
    o = _k4(src, dst, dinv, g3a, g3b, s3a, s3b, params)
    return o[:, None]


# speedup vs baseline: 285.1936x; 1.3348x over previous
import functools

import jax
import jax.numpy as jnp
from jax import lax
from jax.experimental import pallas as pl
from jax.experimental.pallas import tpu as pltpu
from jax.experimental.pallas import tpu_sc as plsc

f32 = jnp.float32
i32 = jnp.int32

NN = 100000
EE = 6400000
NC = 2
NS = 16
NW = NC * NS
NP = 102400
NSL = NP // NS
EW = EE // NW
CE = 10000
E2 = EE // 2
PW = E2 // NW
CP = 4000
V16 = 16


def _mesh():
    return plsc.VectorSubcoreMesh(core_axis_name="c", subcore_axis_name="s")


def _wid():
    return lax.axis_index("c") * NS + lax.axis_index("s")


def _fill(ref, n, val):
    def body(i, _):
        ref[pl.ds(i * V16, V16)] = jnp.full((V16,), val, f32)
        return 0
    lax.fori_loop(0, n // V16, body, 0)


def _rsqrt16(d):
    ii = plsc.bitcast(d, i32)
    seed = jnp.full((V16,), 0x5F3759DF, i32) - lax.shift_right_arithmetic(
        ii, jnp.full((V16,), 1, i32))
    y = plsc.bitcast(seed, f32)
    for _ in range(3):
        y = y * (1.5 - 0.5 * d * y * y)
    return y


def _deg_body(dst_hbm, degp_hbm, acc_sh, idxb, oneb, zb):
    c = lax.axis_index("c")
    s = lax.axis_index("s")
    sl = pl.ds(s * NSL, NSL)
    _fill(zb, NSL, 0.0)
    pltpu.sync_copy(zb, acc_sh.at[sl])
    _fill(oneb, CE, 1.0)
    plsc.subcore_barrier()
    base = _wid() * EW

    def chunk(i, _):
        pltpu.sync_copy(dst_hbm.at[pl.ds(base + i * CE, CE)], idxb)
        pltpu.sync_copy(oneb, acc_sh.at[idxb], add=True)
        return 0
    lax.fori_loop(0, EW // CE, chunk, 0)
    plsc.subcore_barrier()
    pltpu.sync_copy(acc_sh.at[sl], degp_hbm.at[c, sl])


_deg = pl.kernel(
    _deg_body,
    out_type=jax.ShapeDtypeStruct((NC, NP), f32),
    mesh=_mesh(),
    compiler_params=pltpu.CompilerParams(needs_layout_passes=False),
    scratch_types=[
        pltpu.VMEM_SHARED((NP,), f32),
        pltpu.VMEM((CE,), i32),
        pltpu.VMEM((CE,), f32),
        pltpu.VMEM((NSL,), f32),
    ],
)


def _p1_body(src_hbm, dst_hbm, xt_hbm, degp_hbm, par_hbm,
             dinv_hbm, g1_hbm, s1p_hbm,
             tab_sh, acc_sh, pb, b0, b1, q0, q1, q2, q3, db, gb, ib, jb, vb):
    c = lax.axis_index("c")
    s = lax.axis_index("s")
    sl = pl.ds(s * NSL, NSL)
    pltpu.sync_copy(par_hbm, pb)
    pltpu.sync_copy(degp_hbm.at[0, sl], b0)
    pltpu.sync_copy(degp_hbm.at[1, sl], b1)
    pltpu.sync_copy(xt_hbm.at[0, sl], q0)
    pltpu.sync_copy(xt_hbm.at[1, sl], q1)
    pltpu.sync_copy(xt_hbm.at[2, sl], q2)
    pltpu.sync_copy(xt_hbm.at[3, sl], q3)
    pv = pb[pl.ds(0, 16)]
    w0 = pv[0]
    w1 = pv[1]
    w2 = pv[2]
    w3 = pv[3]

    def nodes(i, _):
        dd = pl.ds(i * V16, V16)
        deg = b0[dd] + b1[dd] + 1.0
        y = _rsqrt16(deg)
        t0 = q0[dd] * w0 + q1[dd] * w1 + q2[dd] * w2 + q3[dd] * w3
        db[dd] = y
        gb[dd] = y * t0
        return 0
    lax.fori_loop(0, NSL // V16, nodes, 0)
    pltpu.sync_copy(db, dinv_hbm.at[sl])
    pltpu.sync_copy(gb, g1_hbm.at[sl])
    pltpu.sync_copy(gb, tab_sh.at[sl])
    _fill(b0, NSL, 0.0)
    pltpu.sync_copy(b0, acc_sh.at[sl])
    plsc.subcore_barrier()
    base = _wid() * EW

    def chunk(i, _):
        off = base + i * CE
        pltpu.sync_copy(src_hbm.at[pl.ds(off, CE)], ib)
        pltpu.sync_copy(dst_hbm.at[pl.ds(off, CE)], jb)
        pltpu.sync_copy(tab_sh.at[ib], vb)
        pltpu.sync_copy(vb, acc_sh.at[jb], add=True)
        return 0
    lax.fori_loop(0, EW // CE, chunk, 0)
    plsc.subcore_barrier()
    pltpu.sync_copy(acc_sh.at[sl], s1p_hbm.at[c, sl])


_k1 = pl.kernel(
    _p1_body,
    out_type=(jax.ShapeDtypeStruct((NP,), f32),
              jax.ShapeDtypeStruct((NP,), f32),
              jax.ShapeDtypeStruct((NC, NP), f32)),
    mesh=_mesh(),
    compiler_params=pltpu.CompilerParams(needs_layout_passes=False),
    scratch_types=[
        pltpu.VMEM_SHARED((NP,), f32),
        pltpu.VMEM_SHARED((NP,), f32),
        pltpu.VMEM((32,), f32),
        pltpu.VMEM((NSL,), f32),
        pltpu.VMEM((NSL,), f32),
        pltpu.VMEM((NSL,), f32),
        pltpu.VMEM((NSL,), f32),
        pltpu.VMEM((NSL,), f32),
        pltpu.VMEM((NSL,), f32),
        pltpu.VMEM((NSL,), f32),
        pltpu.VMEM((NSL,), f32),
        pltpu.VMEM((CE,), i32),
        pltpu.VMEM((CE,), i32),
        pltpu.VMEM((CE,), f32),
    ],
)


def _p2_body(src_hbm, dst_hbm, dinv_hbm, g1_hbm, s1p_hbm, par_hbm,
             g2_hbm, s2p_hbm,
             tab_sh, acc_sh, pb, b0, b1, dq, gq, gb, ib, jb, vb):
    c = lax.axis_index("c")
    s = lax.axis_index("s")
    sl = pl.ds(s * NSL, NSL)
    pltpu.sync_copy(par_hbm, pb)
    pltpu.sync_copy(s1p_hbm.at[0, sl], b0)
    pltpu.sync_copy(s1p_hbm.at[1, sl], b1)
    pltpu.sync_copy(dinv_hbm.at[sl], dq)
    pltpu.sync_copy(g1_hbm.at[sl], gq)
    pv = pb[pl.ds(0, 16)]
    bias1 = pv[4]

    def nodes(i, _):
        dd = pl.ds(i * V16, V16)
        d = dq[dd]
        h1 = jnp.maximum(d * (b0[dd] + b1[dd] + gq[dd]) + bias1, 0.0)
        gb[dd] = d * h1
        return 0
    lax.fori_loop(0, NSL // V16, nodes, 0)
    pltpu.sync_copy(gb, g2_hbm.at[sl])
    pltpu.sync_copy(gb, tab_sh.at[sl])
    _fill(b0, NSL, 0.0)
    pltpu.sync_copy(b0, acc_sh.at[sl])
    plsc.subcore_barrier()
    base = _wid() * EW

    def chunk(i, _):
        off = base + i * CE
        pltpu.sync_copy(src_hbm.at[pl.ds(off, CE)], ib)
        pltpu.sync_copy(dst_hbm.at[pl.ds(off, CE)], jb)
        pltpu.sync_copy(tab_sh.at[ib], vb)
        pltpu.sync_copy(vb, acc_sh.at[jb], add=True)
        return 0
    lax.fori_loop(0, EW // CE, chunk, 0)
    plsc.subcore_barrier()
    pltpu.sync_copy(acc_sh.at[sl], s2p_hbm.at[c, sl])


_k2 = pl.kernel(
    _p2_body,
    out_type=(jax.ShapeDtypeStruct((NP,), f32),
              jax.ShapeDtypeStruct((NC, NP), f32)),
    mesh=_mesh(),
    compiler_params=pltpu.CompilerParams(needs_layout_passes=False),
    scratch_types=[
        pltpu.VMEM_SHARED((NP,), f32),
        pltpu.VMEM_SHARED((NP,), f32),
        pltpu.VMEM((32,), f32),
        pltpu.VMEM((NSL,), f32),
        pltpu.VMEM((NSL,), f32),
        pltpu.VMEM((NSL,), f32),
        pltpu.VMEM((NSL,), f32),
        pltpu.VMEM((NSL,), f32),
        pltpu.VMEM((CE,), i32),
        pltpu.VMEM((CE,), i32),
        pltpu.VMEM((CE,), f32),
    ],
)


def _p3_body(src_hbm, dst_hbm, dinv_hbm, g2_hbm, s2p_hbm, par_hbm,
             g3a_hbm, g3b_hbm, s3a_hbm, s3b_hbm,
             tpk_sh, acca_sh, accb_sh,
             pb, b0, b1, dq, gq, ga, gb2, pkb, ib, jb, wb, va, vb):
    c = lax.axis_index("c")
    s = lax.axis_index("s")
    sl = pl.ds(s * NSL, NSL)
    pltpu.sync_copy(par_hbm, pb)
    pltpu.sync_copy(s2p_hbm.at[0, sl], b0)
    pltpu.sync_copy(s2p_hbm.at[1, sl], b1)
    pltpu.sync_copy(dinv_hbm.at[sl], dq)
    pltpu.sync_copy(g2_hbm.at[sl], gq)
    pv = pb[pl.ds(0, 16)]
    w2a = pv[5]
    w2b = pv[6]
    b2a = pv[7]
    b2b = pv[8]

    def nodes(i, _):
        dd = pl.ds(i * V16, V16)
        d = dq[dd]
        u = d * (b0[dd] + b1[dd] + gq[dd])
        xa = d * jnp.maximum(u * w2a + b2a, 0.0)
        xb = d * jnp.maximum(u * w2b + b2b, 0.0)
        ga[dd] = xa
        gb2[dd] = xb
        pkb[dd] = plsc.bitcast(
            plsc.pack(xa, xb, format=plsc.PackFormat.INTERLEAVED), i32)
        return 0
    lax.fori_loop(0, NSL // V16, nodes, 0)
    pltpu.sync_copy(ga, g3a_hbm.at[sl])
    pltpu.sync_copy(gb2, g3b_hbm.at[sl])
    pltpu.sync_copy(pkb, tpk_sh.at[sl])
    _fill(b0, NSL, 0.0)
    pltpu.sync_copy(b0, acca_sh.at[sl])
    pltpu.sync_copy(b0, accb_sh.at[sl])
    plsc.subcore_barrier()
    base = _wid() * EW

    def chunk(i, _):
        off = base + i * CE
        pltpu.sync_copy(src_hbm.at[pl.ds(off, CE)], ib)
        pltpu.sync_copy(dst_hbm.at[pl.ds(off, CE)], jb)
        pltpu.sync_copy(tpk_sh.at[ib], wb)

        def unpk(j, _):
            dd = pl.ds(j * V16, V16)
            xa, xb = plsc.unpack(plsc.bitcast(wb[dd], jnp.bfloat16),
                                 format=plsc.PackFormat.INTERLEAVED)
            va[dd] = xa
            vb[dd] = xb
            return 0
        lax.fori_loop(0, CE // V16, unpk, 0)
        pltpu.sync_copy(va, acca_sh.at[jb], add=True)
        pltpu.sync_copy(vb, accb_sh.at[jb], add=True)
        return 0
    lax.fori_loop(0, EW // CE, chunk, 0)
    plsc.subcore_barrier()
    pltpu.sync_copy(acca_sh.at[sl], s3a_hbm.at[c, sl])
    pltpu.sync_copy(accb_sh.at[sl], s3b_hbm.at[c, sl])


_k3 = pl.kernel(
    _p3_body,
    out_type=(jax.ShapeDtypeStruct((NP,), f32),
              jax.ShapeDtypeStruct((NP,), f32),
              jax.ShapeDtypeStruct((NC, NP), f32),
              jax.ShapeDtypeStruct((NC, NP), f32)),
    mesh=_mesh(),
    compiler_params=pltpu.CompilerParams(needs_layout_passes=False),
    scratch_types=[
        pltpu.VMEM_SHARED((NP,), i32),
        pltpu.VMEM_SHARED((NP,), f32),
        pltpu.VMEM_SHARED((NP,), f32),
        pltpu.VMEM((32,), f32),
        pltpu.VMEM((NSL,), f32),
        pltpu.VMEM((NSL,), f32),
        pltpu.VMEM((NSL,), f32),
        pltpu.VMEM((NSL,), f32),
        pltpu.VMEM((NSL,), f32),
        pltpu.VMEM((NSL,), f32),
        pltpu.VMEM((NSL,), i32),
        pltpu.VMEM((CE,), i32),
        pltpu.VMEM((CE,), i32),
        pltpu.VMEM((CE,), i32),
        pltpu.VMEM((CE,), f32),
        pltpu.VMEM((CE,), f32),
    ],
)


def _ro_body(src_hbm, dst_hbm, dinv_hbm, g3a_hbm, g3b_hbm,
             s3a_hbm, s3b_hbm, par_hbm, o_hbm,
             tza_sh, tzr_sh, tt_sh,
             pb, dq, a0, a1, e0, e1, gqa, gqb, zab, zrb,
             ib0, jb0, wa0, wr0, ft0, sb0,
             ib1, jb1, wa1, wr1, ft1, sb1,
             semi0, semi1, semg0, semg1, semo):
    s = lax.axis_index("s")
    sl = pl.ds(s * NSL, NSL)
    pltpu.sync_copy(par_hbm, pb)
    pltpu.sync_copy(s3a_hbm.at[0, sl], a0)
    pltpu.sync_copy(s3a_hbm.at[1, sl], a1)
    pltpu.sync_copy(s3b_hbm.at[0, sl], e0)
    pltpu.sync_copy(s3b_hbm.at[1, sl], e1)
    pltpu.sync_copy(dinv_hbm.at[sl], dq)
    pltpu.sync_copy(g3a_hbm.at[sl], gqa)
    pltpu.sync_copy(g3b_hbm.at[sl], gqb)
    pv = pb[pl.ds(0, 16)]
    g00 = pv[9]
    g01 = pv[10]
    g10 = pv[11]
    g11 = pv[12]
    c0 = pv[13]
    c1 = pv[14]
    k0 = pv[15]
    pv2 = pb[pl.ds(16, 16)]
    hasc = pv2[0] > 0.5

    def nodes(i, _):
        dd = pl.ds(i * V16, V16)
        d = dq[dd]
        va = d * (a0[dd] + a1[dd] + gqa[dd])
        vb = d * (e0[dd] + e1[dd] + gqb[dd])
        zab[dd] = plsc.bitcast(
            plsc.pack(va, vb, format=plsc.PackFormat.INTERLEAVED), i32)
        zrb[dd] = plsc.bitcast(
            plsc.pack(g00 * va + g01 * vb + c0, g10 * va + g11 * vb + c1,
                      format=plsc.PackFormat.INTERLEAVED), i32)
        gqa[dd] = c0 * va + c1 * vb + k0
        return 0
    lax.fori_loop(0, NSL // V16, nodes, 0)
    pltpu.sync_copy(zab, tza_sh.at[sl])
    pltpu.sync_copy(zrb, tzr_sh.at[sl])
    pltpu.sync_copy(gqa, tt_sh.at[sl])

    @pl.when(jnp.logical_not(hasc))
    def _():
        _fill(ft0, CP, 0.0)
        _fill(ft1, CP, 0.0)

        def addk(i, _):
            dd = pl.ds(i * V16, V16)
            ft0[dd] = ft0[dd] + k0
            ft1[dd] = ft1[dd] + k0
            return 0
        lax.fori_loop(0, CP // V16, addk, 0)
    plsc.subcore_barrier()
    base = _wid() * PW
    NCH = PW // CP

    sets = ((ib0, jb0, wa0, wr0, ft0, sb0, semi0, semg0),
            (ib1, jb1, wa1, wr1, ft1, sb1, semi1, semg1))

    def issue_idx(off, st):
        ib, jb = st[0], st[1]
        pltpu.async_copy(src_hbm.at[pl.ds(off, CP)], ib, st[6])
        pltpu.async_copy(dst_hbm.at[pl.ds(off, CP)], jb, st[6])

    def wait_idx(st):
        pltpu.make_async_copy(src_hbm.at[pl.ds(0, CP)], st[0], st[6]).wait()
        pltpu.make_async_copy(dst_hbm.at[pl.ds(0, CP)], st[1], st[6]).wait()

    def issue_g(st):
        pltpu.async_copy(tza_sh.at[st[0]], st[2], st[7])
        pltpu.async_copy(tzr_sh.at[st[1]], st[3], st[7])

        @pl.when(hasc)
        def _():
            pltpu.async_copy(tt_sh.at[st[1]], st[4], st[7])

    def wait_g(st):
        pltpu.make_async_copy(tza_sh.at[st[0]], st[2], st[7]).wait()
        pltpu.make_async_copy(tzr_sh.at[st[1]], st[3], st[7]).wait()

        @pl.when(hasc)
        def _():
            pltpu.make_async_copy(tt_sh.at[st[1]], st[4], st[7]).wait()

    s0 = sets[0]
    s1 = sets[1]
    pltpu.sync_copy(src_hbm.at[pl.ds(base, CP)], ib0)
    pltpu.sync_copy(dst_hbm.at[pl.ds(base, CP)], jb0)
    issue_g(s0)
    issue_idx(base + E2, s1)

    def chunk(i, _):
        off_next = base + (i + 1) * CP
        wait_idx(s1)
        issue_g(s1)
        wait_g(s0)

        @pl.when(i < NCH - 1)
        def _():
            issue_idx(off_next, s0)

        def dot1(j, _):
            dd = pl.ds(j * V16, V16)
            va, vb = plsc.unpack(plsc.bitcast(wa0[dd], jnp.bfloat16),
                                 format=plsc.PackFormat.INTERLEAVED)
            r0, r1 = plsc.unpack(plsc.bitcast(wr0[dd], jnp.bfloat16),
                                 format=plsc.PackFormat.INTERLEAVED)
            sb0[dd] = va * r0 + vb * r1 + ft0[dd]
            return 0
        lax.fori_loop(0, CP // V16, dot1, 0)

        @pl.when(i < NCH - 1)
        def _():
            wait_idx(s0)
            issue_g(s0)
        wait_g(s1)

        @pl.when(i < NCH - 1)
        def _():
            issue_idx(off_next + E2, s1)

        @pl.when(i > 0)
        def _():
            pltpu.make_async_copy(sb1, o_hbm.at[pl.ds(0, CP)], semo).wait()

        def dot2(j, _):
            dd = pl.ds(j * V16, V16)
            va, vb = plsc.unpack(plsc.bitcast(wa1[dd], jnp.bfloat16),
                                 format=plsc.PackFormat.INTERLEAVED)
            r0, r1 = plsc.unpack(plsc.bitcast(wr1[dd], jnp.bfloat16),
                                 format=plsc.PackFormat.INTERLEAVED)
            sv = 0.5 * (sb0[dd] + va * r0 + vb * r1 + ft1[dd])
            sb1[dd] = 1.0 / (1.0 + jnp.exp(-sv))
            return 0
        lax.fori_loop(0, CP // V16, dot2, 0)
        pltpu.async_copy(sb1, o_hbm.at[pl.ds(base + i * CP, CP)], semo)
        return 0
    lax.fori_loop(0, NCH, chunk, 0)
    pltpu.make_async_copy(sb1, o_hbm.at[pl.ds(0, CP)], semo).wait()


_k4 = pl.kernel(
    _ro_body,
    out_type=jax.ShapeDtypeStruct((E2,), f32),
    mesh=_mesh(),
    compiler_params=pltpu.CompilerParams(needs_layout_passes=False),
    scratch_types=[
        pltpu.VMEM_SHARED((NP,), i32),
        pltpu.VMEM_SHARED((NP,), i32),
        pltpu.VMEM_SHARED((NP,), f32),
        pltpu.VMEM((32,), f32),
        pltpu.VMEM((NSL,), f32),
        pltpu.VMEM((NSL,), f32),
        pltpu.VMEM((NSL,), f32),
        pltpu.VMEM((NSL,), f32),
        pltpu.VMEM((NSL,), f32),
        pltpu.VMEM((NSL,), f32),
        pltpu.VMEM((NSL,), f32),
        pltpu.VMEM((NSL,), i32),
        pltpu.VMEM((NSL,), i32),
        pltpu.VMEM((CP,), i32),
        pltpu.VMEM((CP,), i32),
        pltpu.VMEM((CP,), i32),
        pltpu.VMEM((CP,), i32),
        pltpu.VMEM((CP,), f32),
        pltpu.VMEM((CP,), f32),
        pltpu.VMEM((CP,), i32),
        pltpu.VMEM((CP,), i32),
        pltpu.VMEM((CP,), i32),
        pltpu.VMEM((CP,), i32),
        pltpu.VMEM((CP,), f32),
        pltpu.VMEM((CP,), f32),
        pltpu.SemaphoreType.DMA,
        pltpu.SemaphoreType.DMA,
        pltpu.SemaphoreType.DMA,
        pltpu.SemaphoreType.DMA,
        pltpu.SemaphoreType.DMA,
    ],
)


def kernel(x, edge_index, W1, b1, W2, b2, W3, b3):
    src = edge_index[0]
    dst = edge_index[1]
    xt = jnp.zeros((4, NP), f32).at[:, :NN].set(x.T)
    G = W3 @ W3.T
    cvec = W3 @ b3
    k0 = jnp.dot(b3, b3)
    hasc = jnp.any(cvec != 0).astype(f32)
    params = jnp.concatenate(
        [W1[:, 0], b1, W2[0], b2, G.ravel(), cvec, k0[None], hasc[None],
         jnp.zeros((15,), f32)]).astype(f32)
    degp = _deg(dst)
    dinv, g1, s1p = _k1(src, dst, xt, degp, params)
    g2, s2p = _k2(src, dst, dinv, g1, s1p, params)
    g3a, g3b, s3a, s3b = _k3(src, dst, dinv, g2, s2p, params)
    o = _k4(src, dst, dinv, g3a, g3b, s3a, s3b, params)
    return o[:, None]


# speedup vs baseline: 324.3155x; 1.1372x over previous
import functools

import jax
import jax.numpy as jnp
from jax import lax
from jax.experimental import pallas as pl
from jax.experimental.pallas import tpu as pltpu
from jax.experimental.pallas import tpu_sc as plsc

f32 = jnp.float32
i32 = jnp.int32

NN = 100000
EE = 6400000
NC = 2
NS = 16
NW = NC * NS
NP = 102400
NSL = NP // NS
EW = EE // NW
CE = 10000
E2 = EE // 2
PW = E2 // NW
CP = 4000
CE3 = 4000
V16 = 16


def _mesh():
    return plsc.VectorSubcoreMesh(core_axis_name="c", subcore_axis_name="s")


def _wid():
    return lax.axis_index("c") * NS + lax.axis_index("s")


def _fill(ref, n, val):
    def body(i, _):
        ref[pl.ds(i * V16, V16)] = jnp.full((V16,), val, f32)
        return 0
    lax.fori_loop(0, n // V16, body, 0)


def _rsqrt16(d):
    ii = plsc.bitcast(d, i32)
    seed = jnp.full((V16,), 0x5F3759DF, i32) - lax.shift_right_arithmetic(
        ii, jnp.full((V16,), 1, i32))
    y = plsc.bitcast(seed, f32)
    for _ in range(3):
        y = y * (1.5 - 0.5 * d * y * y)
    return y


def _sweep_hist(dst_hbm, oneb, acc_sh, sets, base, nch, ce):
    (jb0, si0, ss0), (jb1, si1, ss1) = sets

    def i_idx(off, jb, sem):
        pltpu.async_copy(dst_hbm.at[pl.ds(off, ce)], jb, sem)

    def w_idx(jb, sem):
        pltpu.make_async_copy(dst_hbm.at[pl.ds(0, ce)], jb, sem).wait()

    def i_s(jb, sem):
        pltpu.async_copy(oneb, acc_sh.at[jb], sem)

    def w_s(jb, sem):
        pltpu.make_async_copy(oneb, acc_sh.at[jb], sem).wait()

    i_idx(base, jb0, si0)

    def body(p, _):
        q0 = 2 * p
        w_idx(jb0, si0)

        @pl.when(q0 > 0)
        def _():
            w_s(jb1, ss1)

        @pl.when(q0 + 1 < nch)
        def _():
            i_idx(base + (q0 + 1) * ce, jb1, si1)
        i_s(jb0, ss0)

        @pl.when(q0 + 1 < nch)
        def _():
            w_idx(jb1, si1)
            i_s(jb1, ss1)
        w_s(jb0, ss0)

        @pl.when(q0 + 2 < nch)
        def _():
            i_idx(base + (q0 + 2) * ce, jb0, si0)
        return 0
    lax.fori_loop(0, nch // 2, body, 0)
    w_s(jb1, ss1)


def _sweep_gs(src_hbm, dst_hbm, tab_sh, acc_sh, sets, base, nch, ce):
    (ib0, jb0, vb0, si0, sg0, ss0), (ib1, jb1, vb1, si1, sg1, ss1) = sets

    def i_idx(off, ib, jb, sem):
        pltpu.async_copy(src_hbm.at[pl.ds(off, ce)], ib, sem)
        pltpu.async_copy(dst_hbm.at[pl.ds(off, ce)], jb, sem)

    def w_idx(ib, jb, sem):
        pltpu.make_async_copy(src_hbm.at[pl.ds(0, ce)], ib, sem).wait()
        pltpu.make_async_copy(dst_hbm.at[pl.ds(0, ce)], jb, sem).wait()

    def i_g(ib, vb, sem):
        pltpu.async_copy(tab_sh.at[ib], vb, sem)

    def w_g(ib, vb, sem):
        pltpu.make_async_copy(tab_sh.at[ib], vb, sem).wait()

    def i_s(jb, vb, sem):
        pltpu.async_copy(vb, acc_sh.at[jb], sem, add=True)

    def w_s(jb, vb, sem):
        pltpu.make_async_copy(vb, acc_sh.at[jb], sem).wait()

    pltpu.sync_copy(src_hbm.at[pl.ds(base, ce)], ib0)
    pltpu.sync_copy(dst_hbm.at[pl.ds(base, ce)], jb0)
    i_g(ib0, vb0, sg0)

    def body(p, _):
        q0 = 2 * p
        w_g(ib0, vb0, sg0)

        @pl.when(q0 > 0)
        def _():
            w_s(jb1, vb1, ss1)

        @pl.when(q0 + 1 < nch)
        def _():
            i_idx(base + (q0 + 1) * ce, ib1, jb1, si1)
        i_s(jb0, vb0, ss0)

        @pl.when(q0 + 1 < nch)
        def _():
            w_idx(ib1, jb1, si1)
            i_g(ib1, vb1, sg1)
            w_g(ib1, vb1, sg1)
            i_s(jb1, vb1, ss1)
        w_s(jb0, vb0, ss0)

        @pl.when(q0 + 2 < nch)
        def _():
            i_idx(base + (q0 + 2) * ce, ib0, jb0, si0)
            w_idx(ib0, jb0, si0)
            i_g(ib0, vb0, sg0)
        return 0
    lax.fori_loop(0, nch // 2, body, 0)
    w_s(jb1, vb1, ss1)


def _sweep_gs2(src_hbm, dst_hbm, tpk_sh, acca_sh, accb_sh,
               sets, base, nch, ce):
    (ib0, jb0, wb0, va0, vb0, si0, sg0, ss0), \
        (ib1, jb1, wb1, va1, vb1, si1, sg1, ss1) = sets

    def i_idx(off, ib, jb, sem):
        pltpu.async_copy(src_hbm.at[pl.ds(off, ce)], ib, sem)
        pltpu.async_copy(dst_hbm.at[pl.ds(off, ce)], jb, sem)

    def w_idx(ib, jb, sem):
        pltpu.make_async_copy(src_hbm.at[pl.ds(0, ce)], ib, sem).wait()
        pltpu.make_async_copy(dst_hbm.at[pl.ds(0, ce)], jb, sem).wait()

    def i_g(ib, wb, sem):
        pltpu.async_copy(tpk_sh.at[ib], wb, sem)

    def w_g(ib, wb, sem):
        pltpu.make_async_copy(tpk_sh.at[ib], wb, sem).wait()

    def unpk(wb, va, vb):
        def u(j, _):
            dd = pl.ds(j * V16, V16)
            xa, xb = plsc.unpack(plsc.bitcast(wb[dd], jnp.bfloat16),
                                 format=plsc.PackFormat.INTERLEAVED)
            va[dd] = xa
            vb[dd] = xb
            return 0
        lax.fori_loop(0, ce // V16, u, 0)

    def i_s(jb, va, vb, sem):
        pltpu.async_copy(va, acca_sh.at[jb], sem, add=True)
        pltpu.async_copy(vb, accb_sh.at[jb], sem, add=True)

    def w_s(jb, va, vb, sem):
        pltpu.make_async_copy(va, acca_sh.at[jb], sem).wait()
        pltpu.make_async_copy(vb, accb_sh.at[jb], sem).wait()

    pltpu.sync_copy(src_hbm.at[pl.ds(base, ce)], ib0)
    pltpu.sync_copy(dst_hbm.at[pl.ds(base, ce)], jb0)
    i_g(ib0, wb0, sg0)

    def body(p, _):
        q0 = 2 * p
        w_g(ib0, wb0, sg0)
        unpk(wb0, va0, vb0)

        @pl.when(q0 > 0)
        def _():
            w_s(jb1, va1, vb1, ss1)

        @pl.when(q0 + 1 < nch)
        def _():
            i_idx(base + (q0 + 1) * ce, ib1, jb1, si1)
        i_s(jb0, va0, vb0, ss0)

        @pl.when(q0 + 1 < nch)
        def _():
            w_idx(ib1, jb1, si1)
            i_g(ib1, wb1, sg1)
            w_g(ib1, wb1, sg1)
            unpk(wb1, va1, vb1)
            i_s(jb1, va1, vb1, ss1)
        w_s(jb0, va0, vb0, ss0)

        @pl.when(q0 + 2 < nch)
        def _():
            i_idx(base + (q0 + 2) * ce, ib0, jb0, si0)
            w_idx(ib0, jb0, si0)
            i_g(ib0, wb0, sg0)
        return 0
    lax.fori_loop(0, nch // 2, body, 0)
    w_s(jb1, va1, vb1, ss1)


def _deg_body(dst_hbm, degp_hbm, acc_sh, oneb, zb,
              jb0, jb1, si0, si1, ss0, ss1):
    c = lax.axis_index("c")
    s = lax.axis_index("s")
    sl = pl.ds(s * NSL, NSL)
    _fill(zb, NSL, 0.0)
    pltpu.sync_copy(zb, acc_sh.at[sl])
    _fill(oneb, CE, 1.0)
    plsc.subcore_barrier()
    base = _wid() * EW
    sets = ((jb0, si0, ss0), (jb1, si1, ss1))
    _sweep_hist(dst_hbm, oneb, acc_sh, sets, base, EW // CE, CE)
    plsc.subcore_barrier()
    pltpu.sync_copy(acc_sh.at[sl], degp_hbm.at[c, sl])


_deg = pl.kernel(
    _deg_body,
    out_type=jax.ShapeDtypeStruct((NC, NP), f32),
    mesh=_mesh(),
    compiler_params=pltpu.CompilerParams(needs_layout_passes=False),
    scratch_types=[
        pltpu.VMEM_SHARED((NP,), f32),
        pltpu.VMEM((CE,), f32),
        pltpu.VMEM((NSL,), f32),
        pltpu.VMEM((CE,), i32),
        pltpu.VMEM((CE,), i32),
        pltpu.SemaphoreType.DMA,
        pltpu.SemaphoreType.DMA,
        pltpu.SemaphoreType.DMA,
        pltpu.SemaphoreType.DMA,
    ],
)


def _p1_body(src_hbm, dst_hbm, xt_hbm, degp_hbm, par_hbm,
             dinv_hbm, g1_hbm, s1p_hbm,
             tab_sh, acc_sh, pb, b0, b1, q0, q1, q2, q3, db, gb,
             ib0, jb0, vb0, ib1, jb1, vb1, si0, si1, sg0, sg1, ss0, ss1):
    c = lax.axis_index("c")
    s = lax.axis_index("s")
    sl = pl.ds(s * NSL, NSL)
    pltpu.sync_copy(par_hbm, pb)
    pltpu.sync_copy(degp_hbm.at[0, sl], b0)
    pltpu.sync_copy(degp_hbm.at[1, sl], b1)
    pltpu.sync_copy(xt_hbm.at[0, sl], q0)
    pltpu.sync_copy(xt_hbm.at[1, sl], q1)
    pltpu.sync_copy(xt_hbm.at[2, sl], q2)
    pltpu.sync_copy(xt_hbm.at[3, sl], q3)
    pv = pb[pl.ds(0, 16)]
    w0 = pv[0]
    w1 = pv[1]
    w2 = pv[2]
    w3 = pv[3]

    def nodes(i, _):
        dd = pl.ds(i * V16, V16)
        deg = b0[dd] + b1[dd] + 1.0
        y = _rsqrt16(deg)
        t0 = q0[dd] * w0 + q1[dd] * w1 + q2[dd] * w2 + q3[dd] * w3
        db[dd] = y
        gb[dd] = y * t0
        return 0
    lax.fori_loop(0, NSL // V16, nodes, 0)
    pltpu.sync_copy(db, dinv_hbm.at[sl])
    pltpu.sync_copy(gb, g1_hbm.at[sl])
    pltpu.sync_copy(gb, tab_sh.at[sl])
    _fill(b0, NSL, 0.0)
    pltpu.sync_copy(b0, acc_sh.at[sl])
    plsc.subcore_barrier()
    base = _wid() * EW
    _sweep_gs(src_hbm, dst_hbm, tab_sh, acc_sh,
              ((ib0, jb0, vb0, si0, sg0, ss0), (ib1, jb1, vb1, si1, sg1, ss1)),
              base, EW // CE, CE)
    plsc.subcore_barrier()
    pltpu.sync_copy(acc_sh.at[sl], s1p_hbm.at[c, sl])


_k1 = pl.kernel(
    _p1_body,
    out_type=(jax.ShapeDtypeStruct((NP,), f32),
              jax.ShapeDtypeStruct((NP,), f32),
              jax.ShapeDtypeStruct((NC, NP), f32)),
    mesh=_mesh(),
    compiler_params=pltpu.CompilerParams(needs_layout_passes=False),
    scratch_types=[
        pltpu.VMEM_SHARED((NP,), f32),
        pltpu.VMEM_SHARED((NP,), f32),
        pltpu.VMEM((32,), f32),
        pltpu.VMEM((NSL,), f32),
        pltpu.VMEM((NSL,), f32),
        pltpu.VMEM((NSL,), f32),
        pltpu.VMEM((NSL,), f32),
        pltpu.VMEM((NSL,), f32),
        pltpu.VMEM((NSL,), f32),
        pltpu.VMEM((NSL,), f32),
        pltpu.VMEM((NSL,), f32),
        pltpu.VMEM((CE,), i32),
        pltpu.VMEM((CE,), i32),
        pltpu.VMEM((CE,), f32),
        pltpu.VMEM((CE,), i32),
        pltpu.VMEM((CE,), i32),
        pltpu.VMEM((CE,), f32),
        pltpu.SemaphoreType.DMA,
        pltpu.SemaphoreType.DMA,
        pltpu.SemaphoreType.DMA,
        pltpu.SemaphoreType.DMA,
        pltpu.SemaphoreType.DMA,
        pltpu.SemaphoreType.DMA,
    ],
)


def _p2_body(src_hbm, dst_hbm, dinv_hbm, g1_hbm, s1p_hbm, par_hbm,
             g2_hbm, s2p_hbm,
             tab_sh, acc_sh, pb, b0, b1, dq, gq, gb,
             ib0, jb0, vb0, ib1, jb1, vb1, si0, si1, sg0, sg1, ss0, ss1):
    c = lax.axis_index("c")
    s = lax.axis_index("s")
    sl = pl.ds(s * NSL, NSL)
    pltpu.sync_copy(par_hbm, pb)
    pltpu.sync_copy(s1p_hbm.at[0, sl], b0)
    pltpu.sync_copy(s1p_hbm.at[1, sl], b1)
    pltpu.sync_copy(dinv_hbm.at[sl], dq)
    pltpu.sync_copy(g1_hbm.at[sl], gq)
    pv = pb[pl.ds(0, 16)]
    bias1 = pv[4]

    def nodes(i, _):
        dd = pl.ds(i * V16, V16)
        d = dq[dd]
        h1 = jnp.maximum(d * (b0[dd] + b1[dd] + gq[dd]) + bias1, 0.0)
        gb[dd] = d * h1
        return 0
    lax.fori_loop(0, NSL // V16, nodes, 0)
    pltpu.sync_copy(gb, g2_hbm.at[sl])
    pltpu.sync_copy(gb, tab_sh.at[sl])
    _fill(b0, NSL, 0.0)
    pltpu.sync_copy(b0, acc_sh.at[sl])
    plsc.subcore_barrier()
    base = _wid() * EW
    _sweep_gs(src_hbm, dst_hbm, tab_sh, acc_sh,
              ((ib0, jb0, vb0, si0, sg0, ss0), (ib1, jb1, vb1, si1, sg1, ss1)),
              base, EW // CE, CE)
    plsc.subcore_barrier()
    pltpu.sync_copy(acc_sh.at[sl], s2p_hbm.at[c, sl])


_k2 = pl.kernel(
    _p2_body,
    out_type=(jax.ShapeDtypeStruct((NP,), f32),
              jax.ShapeDtypeStruct((NC, NP), f32)),
    mesh=_mesh(),
    compiler_params=pltpu.CompilerParams(needs_layout_passes=False),
    scratch_types=[
        pltpu.VMEM_SHARED((NP,), f32),
        pltpu.VMEM_SHARED((NP,), f32),
        pltpu.VMEM((32,), f32),
        pltpu.VMEM((NSL,), f32),
        pltpu.VMEM((NSL,), f32),
        pltpu.VMEM((NSL,), f32),
        pltpu.VMEM((NSL,), f32),
        pltpu.VMEM((NSL,), f32),
        pltpu.VMEM((CE,), i32),
        pltpu.VMEM((CE,), i32),
        pltpu.VMEM((CE,), f32),
        pltpu.VMEM((CE,), i32),
        pltpu.VMEM((CE,), i32),
        pltpu.VMEM((CE,), f32),
        pltpu.SemaphoreType.DMA,
        pltpu.SemaphoreType.DMA,
        pltpu.SemaphoreType.DMA,
        pltpu.SemaphoreType.DMA,
        pltpu.SemaphoreType.DMA,
        pltpu.SemaphoreType.DMA,
    ],
)


def _p3_body(src_hbm, dst_hbm, dinv_hbm, g2_hbm, s2p_hbm, par_hbm,
             g3a_hbm, g3b_hbm, s3a_hbm, s3b_hbm,
             tpk_sh, acca_sh, accb_sh,
             pb, b0, b1, dq, gq, ga, gb2, pkb,
             ib0, jb0, wb0, va0, vb0, ib1, jb1, wb1, va1, vb1,
             si0, si1, sg0, sg1, ss0, ss1):
    c = lax.axis_index("c")
    s = lax.axis_index("s")
    sl = pl.ds(s * NSL, NSL)
    pltpu.sync_copy(par_hbm, pb)
    pltpu.sync_copy(s2p_hbm.at[0, sl], b0)
    pltpu.sync_copy(s2p_hbm.at[1, sl], b1)
    pltpu.sync_copy(dinv_hbm.at[sl], dq)
    pltpu.sync_copy(g2_hbm.at[sl], gq)
    pv = pb[pl.ds(0, 16)]
    w2a = pv[5]
    w2b = pv[6]
    b2a = pv[7]
    b2b = pv[8]

    def nodes(i, _):
        dd = pl.ds(i * V16, V16)
        d = dq[dd]
        u = d * (b0[dd] + b1[dd] + gq[dd])
        xa = d * jnp.maximum(u * w2a + b2a, 0.0)
        xb = d * jnp.maximum(u * w2b + b2b, 0.0)
        ga[dd] = xa
        gb2[dd] = xb
        pkb[dd] = plsc.bitcast(
            plsc.pack(xa, xb, format=plsc.PackFormat.INTERLEAVED), i32)
        return 0
    lax.fori_loop(0, NSL // V16, nodes, 0)
    pltpu.sync_copy(ga, g3a_hbm.at[sl])
    pltpu.sync_copy(gb2, g3b_hbm.at[sl])
    pltpu.sync_copy(pkb, tpk_sh.at[sl])
    _fill(b0, NSL, 0.0)
    pltpu.sync_copy(b0, acca_sh.at[sl])
    pltpu.sync_copy(b0, accb_sh.at[sl])
    plsc.subcore_barrier()
    base = _wid() * EW
    _sweep_gs2(src_hbm, dst_hbm, tpk_sh, acca_sh, accb_sh,
               ((ib0, jb0, wb0, va0, vb0, si0, sg0, ss0),
                (ib1, jb1, wb1, va1, vb1, si1, sg1, ss1)),
               base, EW // CE3, CE3)
    plsc.subcore_barrier()
    pltpu.sync_copy(acca_sh.at[sl], s3a_hbm.at[c, sl])
    pltpu.sync_copy(accb_sh.at[sl], s3b_hbm.at[c, sl])


_k3 = pl.kernel(
    _p3_body,
    out_type=(jax.ShapeDtypeStruct((NP,), f32),
              jax.ShapeDtypeStruct((NP,), f32),
              jax.ShapeDtypeStruct((NC, NP), f32),
              jax.ShapeDtypeStruct((NC, NP), f32)),
    mesh=_mesh(),
    compiler_params=pltpu.CompilerParams(needs_layout_passes=False),
    scratch_types=[
        pltpu.VMEM_SHARED((NP,), i32),
        pltpu.VMEM_SHARED((NP,), f32),
        pltpu.VMEM_SHARED((NP,), f32),
        pltpu.VMEM((32,), f32),
        pltpu.VMEM((NSL,), f32),
        pltpu.VMEM((NSL,), f32),
        pltpu.VMEM((NSL,), f32),
        pltpu.VMEM((NSL,), f32),
        pltpu.VMEM((NSL,), f32),
        pltpu.VMEM((NSL,), f32),
        pltpu.VMEM((NSL,), i32),
        pltpu.VMEM((CE3,), i32),
        pltpu.VMEM((CE3,), i32),
        pltpu.VMEM((CE3,), i32),
        pltpu.VMEM((CE3,), f32),
        pltpu.VMEM((CE3,), f32),
        pltpu.VMEM((CE3,), i32),
        pltpu.VMEM((CE3,), i32),
        pltpu.VMEM((CE3,), i32),
        pltpu.VMEM((CE3,), f32),
        pltpu.VMEM((CE3,), f32),
        pltpu.SemaphoreType.DMA,
        pltpu.SemaphoreType.DMA,
        pltpu.SemaphoreType.DMA,
        pltpu.SemaphoreType.DMA,
        pltpu.SemaphoreType.DMA,
        pltpu.SemaphoreType.DMA,
    ],
)


def _ro_body(src_hbm, dst_hbm, dinv_hbm, g3a_hbm, g3b_hbm,
             s3a_hbm, s3b_hbm, par_hbm, o_hbm,
             tza_sh, tzr_sh, tt_sh,
             pb, dq, a0, a1, e0, e1, gqa, gqb, zab, zrb,
             ib0, jb0, wa0, wr0, ft0, sb0,
             ib1, jb1, wa1, wr1, ft1, sb1,
             semi0, semi1, semg0, semg1, semo):
    s = lax.axis_index("s")
    sl = pl.ds(s * NSL, NSL)
    pltpu.sync_copy(par_hbm, pb)
    pltpu.sync_copy(s3a_hbm.at[0, sl], a0)
    pltpu.sync_copy(s3a_hbm.at[1, sl], a1)
    pltpu.sync_copy(s3b_hbm.at[0, sl], e0)
    pltpu.sync_copy(s3b_hbm.at[1, sl], e1)
    pltpu.sync_copy(dinv_hbm.at[sl], dq)
    pltpu.sync_copy(g3a_hbm.at[sl], gqa)
    pltpu.sync_copy(g3b_hbm.at[sl], gqb)
    pv = pb[pl.ds(0, 16)]
    g00 = pv[9]
    g01 = pv[10]
    g10 = pv[11]
    g11 = pv[12]
    c0 = pv[13]
    c1 = pv[14]
    k0 = pv[15]
    pv2 = pb[pl.ds(16, 16)]
    hasc = pv2[0] > 0.5

    def nodes(i, _):
        dd = pl.ds(i * V16, V16)
        d = dq[dd]
        va = d * (a0[dd] + a1[dd] + gqa[dd])
        vb = d * (e0[dd] + e1[dd] + gqb[dd])
        zab[dd] = plsc.bitcast(
            plsc.pack(va, vb, format=plsc.PackFormat.INTERLEAVED), i32)
        zrb[dd] = plsc.bitcast(
            plsc.pack(g00 * va + g01 * vb + c0, g10 * va + g11 * vb + c1,
                      format=plsc.PackFormat.INTERLEAVED), i32)
        gqa[dd] = c0 * va + c1 * vb + k0
        return 0
    lax.fori_loop(0, NSL // V16, nodes, 0)
    pltpu.sync_copy(zab, tza_sh.at[sl])
    pltpu.sync_copy(zrb, tzr_sh.at[sl])
    pltpu.sync_copy(gqa, tt_sh.at[sl])

    @pl.when(jnp.logical_not(hasc))
    def _():
        _fill(ft0, CP, 0.0)
        _fill(ft1, CP, 0.0)

        def addk(i, _):
            dd = pl.ds(i * V16, V16)
            ft0[dd] = ft0[dd] + k0
            ft1[dd] = ft1[dd] + k0
            return 0
        lax.fori_loop(0, CP // V16, addk, 0)
    plsc.subcore_barrier()
    base = _wid() * PW
    NCH = PW // CP

    sets = ((ib0, jb0, wa0, wr0, ft0, sb0, semi0, semg0),
            (ib1, jb1, wa1, wr1, ft1, sb1, semi1, semg1))

    def issue_idx(off, st):
        ib, jb = st[0], st[1]
        pltpu.async_copy(src_hbm.at[pl.ds(off, CP)], ib, st[6])
        pltpu.async_copy(dst_hbm.at[pl.ds(off, CP)], jb, st[6])

    def wait_idx(st):
        pltpu.make_async_copy(src_hbm.at[pl.ds(0, CP)], st[0], st[6]).wait()
        pltpu.make_async_copy(dst_hbm.at[pl.ds(0, CP)], st[1], st[6]).wait()

    def issue_g(st):
        pltpu.async_copy(tza_sh.at[st[0]], st[2], st[7])
        pltpu.async_copy(tzr_sh.at[st[1]], st[3], st[7])

        @pl.when(hasc)
        def _():
            pltpu.async_copy(tt_sh.at[st[1]], st[4], st[7])

    def wait_g(st):
        pltpu.make_async_copy(tza_sh.at[st[0]], st[2], st[7]).wait()
        pltpu.make_async_copy(tzr_sh.at[st[1]], st[3], st[7]).wait()

        @pl.when(hasc)
        def _():
            pltpu.make_async_copy(tt_sh.at[st[1]], st[4], st[7]).wait()

    s0 = sets[0]
    s1 = sets[1]
    pltpu.sync_copy(src_hbm.at[pl.ds(base, CP)], ib0)
    pltpu.sync_copy(dst_hbm.at[pl.ds(base, CP)], jb0)
    issue_g(s0)
    issue_idx(base + E2, s1)

    def chunk(i, _):
        off_next = base + (i + 1) * CP
        wait_idx(s1)
        issue_g(s1)
        wait_g(s0)

        @pl.when(i < NCH - 1)
        def _():
            issue_idx(off_next, s0)

        def dot1(j, _):
            dd = pl.ds(j * V16, V16)
            va, vb = plsc.unpack(plsc.bitcast(wa0[dd], jnp.bfloat16),
                                 format=plsc.PackFormat.INTERLEAVED)
            r0, r1 = plsc.unpack(plsc.bitcast(wr0[dd], jnp.bfloat16),
                                 format=plsc.PackFormat.INTERLEAVED)
            sb0[dd] = va * r0 + vb * r1 + ft0[dd]
            return 0
        lax.fori_loop(0, CP // V16, dot1, 0)

        @pl.when(i < NCH - 1)
        def _():
            wait_idx(s0)
            issue_g(s0)
        wait_g(s1)

        @pl.when(i < NCH - 1)
        def _():
            issue_idx(off_next + E2, s1)

        @pl.when(i > 0)
        def _():
            pltpu.make_async_copy(sb1, o_hbm.at[pl.ds(0, CP)], semo).wait()

        def dot2(j, _):
            dd = pl.ds(j * V16, V16)
            va, vb = plsc.unpack(plsc.bitcast(wa1[dd], jnp.bfloat16),
                                 format=plsc.PackFormat.INTERLEAVED)
            r0, r1 = plsc.unpack(plsc.bitcast(wr1[dd], jnp.bfloat16),
                                 format=plsc.PackFormat.INTERLEAVED)
            sv = 0.5 * (sb0[dd] + va * r0 + vb * r1 + ft1[dd])
            sb1[dd] = 1.0 / (1.0 + jnp.exp(-sv))
            return 0
        lax.fori_loop(0, CP // V16, dot2, 0)
        pltpu.async_copy(sb1, o_hbm.at[pl.ds(base + i * CP, CP)], semo)
        return 0
    lax.fori_loop(0, NCH, chunk, 0)
    pltpu.make_async_copy(sb1, o_hbm.at[pl.ds(0, CP)], semo).wait()


_k4 = pl.kernel(
    _ro_body,
    out_type=jax.ShapeDtypeStruct((E2,), f32),
    mesh=_mesh(),
    compiler_params=pltpu.CompilerParams(needs_layout_passes=False),
    scratch_types=[
        pltpu.VMEM_SHARED((NP,), i32),
        pltpu.VMEM_SHARED((NP,), i32),
        pltpu.VMEM_SHARED((NP,), f32),
        pltpu.VMEM((32,), f32),
        pltpu.VMEM((NSL,), f32),
        pltpu.VMEM((NSL,), f32),
        pltpu.VMEM((NSL,), f32),
        pltpu.VMEM((NSL,), f32),
        pltpu.VMEM((NSL,), f32),
        pltpu.VMEM((NSL,), f32),
        pltpu.VMEM((NSL,), f32),
        pltpu.VMEM((NSL,), i32),
        pltpu.VMEM((NSL,), i32),
        pltpu.VMEM((CP,), i32),
        pltpu.VMEM((CP,), i32),
        pltpu.VMEM((CP,), i32),
        pltpu.VMEM((CP,), i32),
        pltpu.VMEM((CP,), f32),
        pltpu.VMEM((CP,), f32),
        pltpu.VMEM((CP,), i32),
        pltpu.VMEM((CP,), i32),
        pltpu.VMEM((CP,), i32),
        pltpu.VMEM((CP,), i32),
        pltpu.VMEM((CP,), f32),
        pltpu.VMEM((CP,), f32),
        pltpu.SemaphoreType.DMA,
        pltpu.SemaphoreType.DMA,
        pltpu.SemaphoreType.DMA,
        pltpu.SemaphoreType.DMA,
        pltpu.SemaphoreType.DMA,
    ],
)


def kernel(x, edge_index, W1, b1, W2, b2, W3, b3):
    src = edge_index[0]
    dst = edge_index[1]
    xt = jnp.zeros((4, NP), f32).at[:, :NN].set(x.T)
    G = W3 @ W3.T
    cvec = W3 @ b3
    k0 = jnp.dot(b3, b3)
    hasc = jnp.any(cvec != 0).astype(f32)
    params = jnp.concatenate(
        [W1[:, 0], b1, W2[0], b2, G.ravel(), cvec, k0[None], hasc[None],
         jnp.zeros((15,), f32)]).astype(f32)
    degp = _deg(dst)
    dinv, g1, s1p = _k1(src, dst, xt, degp, params)
    g2, s2p = _k2(src, dst, dinv, g1, s1p, params)
    g3a, g3b, s3a, s3b = _k3(src, dst, dinv, g2, s2p, params)
    o = _k4(src, dst, dinv, g3a, g3b, s3a, s3b, params)
    return o[:, None]
